# baseline (speedup 1.0000x reference)
import functools

import jax
import jax.numpy as jnp
from jax import lax
from jax.experimental import pallas as pl
from jax.experimental.pallas import tpu as pltpu
from jax.experimental.pallas import tpu_sc as plsc

C = 512
B = 16
L = 2048
D = 64
KEEP = 64
H = 128

NC = 2
NS = 16
LANES = 16
NW = NC * NS
CPW = C // NW
ROWS_PW = CPW * KEEP
STEPS = L // (4 * LANES)

GPLANES = 8
TSTEPS = KEEP // GPLANES


def _sc_pack(mask, batch_idx, kflat):
    mesh = plsc.VectorSubcoreMesh(core_axis_name="c", subcore_axis_name="s")

    @functools.partial(
        pl.kernel,
        out_type=(
            jax.ShapeDtypeStruct((KEEP * C, D), jnp.float32),
            jax.ShapeDtypeStruct((C,), jnp.int32),
        ),
        mesh=mesh,
        compiler_params=pltpu.CompilerParams(
            needs_layout_passes=False, use_tc_tiling_on_sc=False),
        scratch_types=[
            pltpu.VMEM((CPW, L), jnp.int8),
            pltpu.VMEM((ROWS_PW,), jnp.int32),
            pltpu.VMEM((ROWS_PW,), jnp.int32),
            pltpu.VMEM((CPW,), jnp.int32),
            pltpu.VMEM((CPW,), jnp.int32),
            pltpu.VMEM((ROWS_PW, D), jnp.float32),
            pltpu.SemaphoreType.DMA,
            pltpu.SemaphoreType.DMA,
        ],
    )
    def sc_kernel(mask_hbm, bidx_hbm, kflat_hbm, out_hbm, cnt_hbm,
                  mrow, idxv, idxw, bvec, cntv, rows, sem, sem2):
        wid = lax.axis_index("s") * NC + lax.axis_index("c")
        base_chain = wid * CPW
        mask_cp = pltpu.async_copy(
            mask_hbm.at[pl.ds(base_chain, CPW)], mrow, sem2)
        pltpu.sync_copy(bidx_hbm.at[pl.ds(base_chain, CPW)], bvec)

        iota = lax.iota(jnp.int32, LANES)

        pad_base = wid * ROWS_PW
        for i in range(CPW):
            for jj in range(KEEP // LANES):
                j = jj * LANES + iota
                idxv[pl.ds(i * KEEP + jj * LANES, LANES)] = (
                    pad_base + i * KEEP + j)
                idxw[pl.ds(i * KEEP + jj * LANES, LANES)] = (
                    j * C + base_chain + i)

        mask_cp.wait()

        gathers = []
        outs = []
        for i in range(CPW):
            bvals = bvec[...]
            bl = jnp.sum(jnp.where(iota == i, bvals, 0)) * L

            def cond(sc):
                step, cnt = sc
                return jnp.logical_and(step < STEPS, cnt < KEEP)

            def body(sc):
                step, cnt = sc
                v = plsc.bitcast(
                    mrow[i, pl.ds(step * 4 * LANES, 4 * LANES)], jnp.int32)
                c0 = v & 1
                c1 = (v >> 8) & 1
                c2 = (v >> 16) & 1
                c3 = (v >> 24) & 1
                t = c0 + c1 + c2 + c3
                rbase = plsc.cumsum(t) + cnt - t
                pos0 = bl + step * (4 * LANES) + iota * 4
                s = rbase
                for j, c in enumerate((c0, c1, c2, c3)):
                    rank = s + c
                    valid = jnp.logical_and(c > 0, rank <= KEEP)
                    plsc.store_scatter(
                        idxv, [i * KEEP + rank - 1], pos0 + j, mask=valid)
                    s = rank
                return step + 1, cnt + jnp.sum(t)

            _, cnt = lax.while_loop(
                cond, body, (jnp.int32(0), jnp.int32(0)))
            cnt = jnp.minimum(cnt, KEEP)
            plsc.store_scatter(
                cntv,
                [jnp.full((LANES,), i, jnp.int32)],
                jnp.full((LANES,), cnt, jnp.int32),
                mask=iota == 0,
            )
            gathers.append(pltpu.async_copy(
                kflat_hbm.at[idxv.at[pl.ds(i * KEEP, KEEP)]],
                rows.at[pl.ds(i * KEEP, KEEP)],
                sem,
            ))

        pltpu.sync_copy(cntv, cnt_hbm.at[pl.ds(base_chain, CPW)])

        for i in range(CPW):
            gathers[i].wait()
            outs.append(pltpu.async_copy(
                rows.at[pl.ds(i * KEEP, KEEP)],
                out_hbm.at[idxw.at[pl.ds(i * KEEP, KEEP)]],
                sem2,
            ))
        for cp in outs:
            cp.wait()

    return sc_kernel(mask, batch_idx, kflat)


def _mlp_body(q_ref, p_ref, cnt_ref, count_ref, w1_ref,
              b1_ref, w2_ref, b2_ref, o_ref, buf, sema, semb):
    sems = (sema, semb)
    copies = [None] * TSTEPS
    copies[0] = pltpu.make_async_copy(
        p_ref.at[pl.ds(0, GPLANES * C)], buf.at[0], sems[0])
    copies[0].start()

    logc = jnp.log1p(count_ref[...].astype(jnp.float32))
    acc = (jnp.dot(q_ref[...], w1_ref[0:D, :],
                   preferred_element_type=jnp.float32)
           + logc * w1_ref[D + KEEP * D:D + KEEP * D + 1, :]
           + b1_ref[...])

    cnt = cnt_ref[...]
    for t in range(TSTEPS):
        if t + 1 < TSTEPS:
            copies[t + 1] = pltpu.make_async_copy(
                p_ref.at[pl.ds((t + 1) * GPLANES * C, GPLANES * C)],
                buf.at[(t + 1) % 2], sems[(t + 1) % 2])
            copies[t + 1].start()
        copies[t].wait()
        for j in range(GPLANES):
            sslot = t * GPLANES + j
            keep = (cnt > sslot).astype(jnp.float32)
            pm = buf[t % 2, pl.ds(j * C, C), :] * keep
            acc = acc + jnp.dot(
                pm, w1_ref[pl.ds(D + sslot * D, D), :],
                preferred_element_type=jnp.float32)

    h = 0.5 * acc * (1.0 + lax.erf(acc * 0.7071067811865476))
    o_ref[...] = (jnp.dot(h, w2_ref[...], preferred_element_type=jnp.float32)
                  + b2_ref[...])


def _tc_mlp(q, packed, cnt, count, W1, b1, W2, b2):
    return pl.pallas_call(
        _mlp_body,
        in_specs=[
            pl.BlockSpec(memory_space=pl.MemorySpace.DEFAULT),
            pl.BlockSpec(memory_space=pl.ANY),
            pl.BlockSpec(memory_space=pl.MemorySpace.DEFAULT),
            pl.BlockSpec(memory_space=pl.MemorySpace.DEFAULT),
            pl.BlockSpec(memory_space=pl.MemorySpace.DEFAULT),
            pl.BlockSpec(memory_space=pl.MemorySpace.DEFAULT),
            pl.BlockSpec(memory_space=pl.MemorySpace.DEFAULT),
            pl.BlockSpec(memory_space=pl.MemorySpace.DEFAULT),
        ],
        scratch_shapes=[
            pltpu.VMEM((2, GPLANES * C, D), jnp.float32),
            pltpu.SemaphoreType.DMA,
            pltpu.SemaphoreType.DMA,
        ],
        out_shape=jax.ShapeDtypeStruct((C, 1), jnp.float32),
    )(q, packed, cnt, count, W1, b1, W2, b2)


def kernel(q, k, batch_idx, mask, count, W1, b1, W2, b2):
    kflat = k.reshape(B * L, D)
    packed_rows, cnt = _sc_pack(
        mask.view(jnp.int8), batch_idx.astype(jnp.int32), kflat)
    out = _tc_mlp(
        q, packed_rows,
        cnt.reshape(C, 1),
        count.reshape(C, 1).astype(jnp.int32),
        W1,
        b1.reshape(1, H), W2, b2.reshape(1, 1),
    )
    return out.reshape(C)


# speedup vs baseline: 1.1878x; 1.1878x over previous
import functools

import jax
import jax.numpy as jnp
from jax import lax
from jax.experimental import pallas as pl
from jax.experimental.pallas import tpu as pltpu
from jax.experimental.pallas import tpu_sc as plsc

C = 512
B = 16
L = 2048
D = 64
KEEP = 64
H = 128

NC = 2
NS = 16
LANES = 16
NW = NC * NS
CPW = C // NW
ROWS_PW = CPW * KEEP
LP = L // 4
STEPS = L // (4 * LANES)
GCHUNK = 128


def _sc_scan(mask, batch_idx):
    mesh = plsc.VectorSubcoreMesh(core_axis_name="c", subcore_axis_name="s")

    @functools.partial(
        pl.kernel,
        out_type=(
            jax.ShapeDtypeStruct((C * KEEP,), jnp.int32),
            jax.ShapeDtypeStruct((C,), jnp.int32),
        ),
        mesh=mesh,
        compiler_params=pltpu.CompilerParams(
            needs_layout_passes=False, use_tc_tiling_on_sc=False),
        scratch_types=[
            pltpu.VMEM((CPW, L), jnp.int8),
            pltpu.VMEM((ROWS_PW,), jnp.int32),
            pltpu.VMEM((CPW,), jnp.int32),
            pltpu.VMEM((CPW,), jnp.int32),
            pltpu.SemaphoreType.DMA,
        ],
    )
    def scan_kernel(mask_hbm, bidx_hbm, idx_hbm, cnt_hbm,
                    mrow, idxv, bvec, cntv, sem):
        wid = lax.axis_index("s") * NC + lax.axis_index("c")
        base_chain = wid * CPW
        mask_cp = pltpu.async_copy(
            mask_hbm.at[pl.ds(base_chain, CPW)], mrow, sem)
        pltpu.sync_copy(bidx_hbm.at[pl.ds(base_chain, CPW)], bvec)

        iota = lax.iota(jnp.int32, LANES)

        pad_base = wid * ROWS_PW
        for jj in range(ROWS_PW // LANES):
            idxv[pl.ds(jj * LANES, LANES)] = pad_base + jj * LANES + iota

        mask_cp.wait()

        for i in range(CPW):
            bvals = bvec[...]
            bl = jnp.sum(jnp.where(iota == i, bvals, 0)) * L

            def cond(sc):
                step, cnt = sc
                return jnp.logical_and(step < STEPS, cnt < KEEP)

            def body(sc):
                step, cnt = sc
                v = plsc.bitcast(
                    mrow[i, pl.ds(step * 4 * LANES, 4 * LANES)], jnp.int32)
                c0 = v & 1
                c1 = (v >> 8) & 1
                c2 = (v >> 16) & 1
                c3 = (v >> 24) & 1
                t = c0 + c1 + c2 + c3
                rbase = plsc.cumsum(t) + cnt - t
                pos0 = bl + step * (4 * LANES) + iota * 4
                s = rbase
                for j, c in enumerate((c0, c1, c2, c3)):
                    rank = s + c
                    valid = jnp.logical_and(c > 0, rank <= KEEP)
                    plsc.store_scatter(
                        idxv, [i * KEEP + rank - 1], pos0 + j, mask=valid)
                    s = rank
                return step + 1, cnt + jnp.sum(t)

            _, cnt = lax.while_loop(
                cond, body, (jnp.int32(0), jnp.int32(0)))
            cnt = jnp.minimum(cnt, KEEP)
            plsc.store_scatter(
                cntv,
                [jnp.full((LANES,), i, jnp.int32)],
                jnp.full((LANES,), cnt, jnp.int32),
                mask=iota == 0,
            )

        pltpu.sync_copy(cntv, cnt_hbm.at[pl.ds(base_chain, CPW)])
        pltpu.sync_copy(idxv, idx_hbm.at[pl.ds(wid * ROWS_PW, ROWS_PW)])

    return scan_kernel(mask, batch_idx)


def _sc_gather(idx_all, kflat):
    mesh = plsc.VectorSubcoreMesh(core_axis_name="c", subcore_axis_name="s")

    @functools.partial(
        pl.kernel,
        out_type=jax.ShapeDtypeStruct((C * KEEP, D), jnp.float32),
        mesh=mesh,
        compiler_params=pltpu.CompilerParams(
            needs_layout_passes=False, use_tc_tiling_on_sc=False),
        scratch_types=[
            pltpu.VMEM((ROWS_PW,), jnp.int32),
            pltpu.VMEM((ROWS_PW, D), jnp.float32),
            pltpu.SemaphoreType.DMA,
            pltpu.SemaphoreType.DMA,
        ],
    )
    def gather_kernel(idx_hbm, kflat_hbm, out_hbm, idxv, rows, sem, sem2):
        wid = lax.axis_index("s") * NC + lax.axis_index("c")
        base = wid * ROWS_PW
        pltpu.sync_copy(idx_hbm.at[pl.ds(base, ROWS_PW)], idxv)

        gathers = []
        outs = []
        for g in range(ROWS_PW // GCHUNK):
            gathers.append(pltpu.async_copy(
                kflat_hbm.at[idxv.at[pl.ds(g * GCHUNK, GCHUNK)]],
                rows.at[pl.ds(g * GCHUNK, GCHUNK)],
                sem,
            ))
        for g in range(ROWS_PW // GCHUNK):
            gathers[g].wait()
            outs.append(pltpu.async_copy(
                rows.at[pl.ds(g * GCHUNK, GCHUNK)],
                out_hbm.at[pl.ds(base + g * GCHUNK, GCHUNK)],
                sem2,
            ))
        for cp in outs:
            cp.wait()

    return gather_kernel(idx_all, kflat)


def _mlp_body(q_ref, p_ref, cnt_ref, count_ref, w1_ref,
              b1_ref, w2_ref, b2_ref, o_ref):
    slot = lax.broadcasted_iota(jnp.int32, (C, KEEP * D), 1) >> 6
    keepm = (slot < cnt_ref[...]).astype(jnp.float32)
    pm = p_ref[...] * keepm
    logc = jnp.log1p(count_ref[...].astype(jnp.float32))
    h = (jnp.dot(q_ref[...], w1_ref[0:D, :],
                 preferred_element_type=jnp.float32)
         + jnp.dot(pm, w1_ref[D:D + KEEP * D, :],
                   preferred_element_type=jnp.float32)
         + logc * w1_ref[D + KEEP * D:D + KEEP * D + 1, :]
         + b1_ref[...])
    h = 0.5 * h * (1.0 + lax.erf(h * 0.7071067811865476))
    o_ref[...] = (jnp.dot(h, w2_ref[...], preferred_element_type=jnp.float32)
                  + b2_ref[...])


def _tc_mlp(q, packed, cnt, count, W1, b1, W2, b2):
    return pl.pallas_call(
        _mlp_body,
        out_shape=jax.ShapeDtypeStruct((C, 1), jnp.float32),
    )(q, packed, cnt, count, W1, b1, W2, b2)


def kernel(q, k, batch_idx, mask, count, W1, b1, W2, b2):
    kflat = k.reshape(B * L, D)
    idx_all, cnt = _sc_scan(mask.view(jnp.int8), batch_idx.astype(jnp.int32))
    packed_rows = _sc_gather(idx_all, kflat)
    packed = packed_rows.reshape(C, KEEP * D)
    out = _tc_mlp(
        q, packed,
        cnt.reshape(C, 1),
        count.reshape(C, 1).astype(jnp.int32),
        W1,
        b1.reshape(1, H), W2, b2.reshape(1, 1),
    )
    return out.reshape(C)
